# Initial kernel scaffold; baseline (speedup 1.0000x reference)
import jax
import jax.numpy as jnp
from jax.experimental import pallas as pl


def kernel(embed_nt0, embed_nt1, edges_e0, edges_e1, edges_e2, labels_e0, labels_e1, labels_e2, w_e0, w_e1, w_e2):
    raise NotImplementedError("write your pallas kernel here")


# speedup vs baseline: 3.1968x; 3.1968x over previous
import functools

import jax
import jax.numpy as jnp
from jax import lax
from jax.experimental import pallas as pl
from jax.experimental.pallas import tpu as pltpu
from jax.experimental.pallas import tpu_sc as plsc

OUT_DIM = 128
N_NODES = 10000
E = 160000
REG_PARAM = 0.0001

NC = 2
NS = 16
NW = NC * NS
B = 128
NBLK = E // B


def _sc_scores(emb0, emb1, s0, d0, s1, d1, s2, d2, w_all):
    mesh = plsc.VectorSubcoreMesh(core_axis_name="c", subcore_axis_name="s")

    @functools.partial(
        pl.kernel,
        mesh=mesh,
        out_type=jax.ShapeDtypeStruct((3 * E,), jnp.float32),
        compiler_params=pltpu.CompilerParams(needs_layout_passes=False),
        scratch_types=[
            pltpu.VMEM((B,), jnp.int32),
            pltpu.VMEM((B,), jnp.int32),
            pltpu.VMEM((B, OUT_DIM), jnp.float32),
            pltpu.VMEM((B, OUT_DIM), jnp.float32),
            pltpu.VMEM((16, 16), jnp.float32),
            pltpu.VMEM((B,), jnp.float32),
            pltpu.VMEM((3, OUT_DIM), jnp.float32),
            pltpu.SemaphoreType.DMA,
            pltpu.SemaphoreType.DMA,
        ],
    )
    def k(emb0_h, emb1_h, s0_h, d0_h, s1_h, d1_h, s2_h, d2_h, w_h, out_h,
          idx_s, idx_o, srows, orows, tmp_t, svec, w_v, sem_s, sem_o):
        wid = lax.axis_index("s") * NC + lax.axis_index("c")
        nblk_w = (NBLK - wid + NW - 1) // NW
        pltpu.sync_copy(w_h, w_v)
        plans = [
            (s0_h, d0_h, emb0_h, emb1_h),
            (s1_h, d1_h, emb1_h, emb0_h),
            (s2_h, d2_h, emb0_h, emb0_h),
        ]
        for et, (sh, oh, st, ot) in enumerate(plans):
            r = [w_v[et, pl.ds(16 * j, 16)] for j in range(8)]

            def block(i, _, sh=sh, oh=oh, st=st, ot=ot, et=et, r=r):
                base = (wid + i * NW) * B
                pltpu.sync_copy(sh.at[pl.ds(base, B)], idx_s)
                pltpu.sync_copy(oh.at[pl.ds(base, B)], idx_o)
                cs = pltpu.async_copy(st.at[idx_s], srows, sem_s)
                co = pltpu.async_copy(ot.at[idx_o], orows, sem_o)
                cs.wait()
                co.wait()

                lanes = lax.iota(jnp.int32, 16)

                def group(g, _):
                    for k in range(16):
                        e = g * 16 + k
                        acc = srows[e, pl.ds(0, 16)] * r[0] \
                            * orows[e, pl.ds(0, 16)]
                        for j in range(1, 8):
                            acc = acc + srows[e, pl.ds(16 * j, 16)] * r[j] \
                                * orows[e, pl.ds(16 * j, 16)]
                        plsc.store_scatter(
                            tmp_t, [lanes, jnp.full((16,), k, jnp.int32)],
                            acc)
                    score = tmp_t[0, :]
                    for i in range(1, 16):
                        score = score + tmp_t[i, :]
                    svec[pl.ds(g * 16, 16)] = score
                    return 0

                lax.fori_loop(0, B // 16, group, 0)
                pltpu.sync_copy(svec, out_h.at[pl.ds(et * E + base, B)])
                return 0

            lax.fori_loop(0, nblk_w, block, 0)

    return k(emb0, emb1, s0, d0, s1, d1, s2, d2, w_all)


def _tc_loss(scores, labels, emb0, emb1, w_all):
    def body(scores_ref, labels_ref, e0_ref, e1_ref, w_ref, out_ref):
        x = scores_ref[...]
        t = labels_ref[...]
        bce = jnp.clip(x, 0.0, None) - x * t + jnp.log1p(jnp.exp(-jnp.abs(x)))
        predict = jnp.sum(bce) / E
        reg = (jnp.mean(e0_ref[...] ** 2) + jnp.mean(e1_ref[...] ** 2)
               + jnp.sum(w_ref[...] ** 2) / OUT_DIM)
        out_ref[...] = jnp.reshape(predict + REG_PARAM * reg, (1, 1))

    return pl.pallas_call(
        body,
        out_shape=jax.ShapeDtypeStruct((1, 1), jnp.float32),
    )(scores, labels, emb0, emb1, w_all)


def kernel(embed_nt0, embed_nt1, edges_e0, edges_e1, edges_e2,
           labels_e0, labels_e1, labels_e2, w_e0, w_e1, w_e2):
    e0 = edges_e0.astype(jnp.int32)
    e1 = edges_e1.astype(jnp.int32)
    e2 = edges_e2.astype(jnp.int32)
    w_all = jnp.stack([w_e0[:, 0], w_e1[:, 0], w_e2[:, 0]])
    labels = jnp.stack([labels_e0, labels_e1, labels_e2])
    scores = _sc_scores(embed_nt0, embed_nt1,
                        e0[:, 0], e0[:, 1], e1[:, 0], e1[:, 1],
                        e2[:, 0], e2[:, 1], w_all)
    scores = jnp.reshape(scores, (3, E))
    return _tc_loss(scores, labels, embed_nt0, embed_nt1, w_all)[0, 0]


# speedup vs baseline: 4.2995x; 1.3449x over previous
import functools

import jax
import jax.numpy as jnp
from jax import lax
from jax.experimental import pallas as pl
from jax.experimental.pallas import tpu as pltpu
from jax.experimental.pallas import tpu_sc as plsc

OUT_DIM = 128
N_NODES = 10000
E = 160000
REG_PARAM = 0.0001

NC = 2
NS = 16
NW = NC * NS
B = 128
NBLK = E // B


def _sc_scores(emb0, emb1, s0, d0, s1, d1, s2, d2, w_all):
    mesh = plsc.VectorSubcoreMesh(core_axis_name="c", subcore_axis_name="s")

    @functools.partial(
        pl.kernel,
        mesh=mesh,
        out_type=jax.ShapeDtypeStruct((3 * E,), jnp.float32),
        compiler_params=pltpu.CompilerParams(needs_layout_passes=False),
        scratch_types=[
            pltpu.VMEM((2, B), jnp.int32),
            pltpu.VMEM((2, B), jnp.int32),
            pltpu.VMEM((2, B, OUT_DIM), jnp.float32),
            pltpu.VMEM((2, B, OUT_DIM), jnp.float32),
            pltpu.VMEM((16, 16), jnp.float32),
            pltpu.VMEM((B,), jnp.float32),
            pltpu.VMEM((3, OUT_DIM), jnp.float32),
            pltpu.SemaphoreType.DMA((2,)),
            pltpu.SemaphoreType.DMA((2,)),
        ],
    )
    def k(emb0_h, emb1_h, s0_h, d0_h, s1_h, d1_h, s2_h, d2_h, w_h, out_h,
          idx_s, idx_o, srows, orows, tmp_t, svec, w_v, sem_s, sem_o):
        wid = lax.axis_index("s") * NC + lax.axis_index("c")
        nblk_w = (NBLK - wid + NW - 1) // NW
        pltpu.sync_copy(w_h, w_v)
        plans = [
            (s0_h, d0_h, emb0_h, emb1_h),
            (s1_h, d1_h, emb1_h, emb0_h),
            (s2_h, d2_h, emb0_h, emb0_h),
        ]
        for et, (sh, oh, st, ot) in enumerate(plans):
            r = [w_v[et, pl.ds(16 * j, 16)] for j in range(8)]

            def fetch(i, buf, sh=sh, oh=oh, st=st, ot=ot):
                base = (wid + i * NW) * B
                pltpu.sync_copy(sh.at[pl.ds(base, B)], idx_s.at[buf])
                pltpu.sync_copy(oh.at[pl.ds(base, B)], idx_o.at[buf])
                pltpu.async_copy(st.at[idx_s.at[buf]], srows.at[buf],
                                 sem_s.at[buf])
                pltpu.async_copy(ot.at[idx_o.at[buf]], orows.at[buf],
                                 sem_o.at[buf])

            fetch(0, 0)

            def block(i, _, sh=sh, oh=oh, st=st, ot=ot, et=et, r=r):
                cur = lax.rem(i, 2)
                nxt = jnp.minimum(i + 1, nblk_w - 1)
                fetch(nxt, 1 - cur)
                pltpu.make_async_copy(
                    st.at[idx_s.at[cur]], srows.at[cur], sem_s.at[cur]
                ).wait()
                pltpu.make_async_copy(
                    ot.at[idx_o.at[cur]], orows.at[cur], sem_o.at[cur]
                ).wait()

                lanes = lax.iota(jnp.int32, 16)

                def group(g, _):
                    for k in range(16):
                        e = g * 16 + k
                        acc = srows[cur, e, pl.ds(0, 16)] * r[0] \
                            * orows[cur, e, pl.ds(0, 16)]
                        for j in range(1, 8):
                            acc = acc \
                                + srows[cur, e, pl.ds(16 * j, 16)] * r[j] \
                                * orows[cur, e, pl.ds(16 * j, 16)]
                        plsc.store_scatter(
                            tmp_t, [lanes, jnp.full((16,), k, jnp.int32)],
                            acc)
                    score = tmp_t[0, :]
                    for i in range(1, 16):
                        score = score + tmp_t[i, :]
                    svec[pl.ds(g * 16, 16)] = score
                    return 0

                lax.fori_loop(0, B // 16, group, 0)
                base = (wid + i * NW) * B
                pltpu.sync_copy(svec, out_h.at[pl.ds(et * E + base, B)])
                return 0

            lax.fori_loop(0, nblk_w, block, 0)
            lastbuf = lax.rem(nblk_w, 2)
            pltpu.make_async_copy(
                st.at[idx_s.at[lastbuf]], srows.at[lastbuf],
                sem_s.at[lastbuf]).wait()
            pltpu.make_async_copy(
                ot.at[idx_o.at[lastbuf]], orows.at[lastbuf],
                sem_o.at[lastbuf]).wait()

    return k(emb0, emb1, s0, d0, s1, d1, s2, d2, w_all)


def _tc_loss(scores, labels, emb0, emb1, w_all):
    def body(scores_ref, labels_ref, e0_ref, e1_ref, w_ref, out_ref):
        x = scores_ref[...]
        t = labels_ref[...]
        bce = jnp.clip(x, 0.0, None) - x * t + jnp.log1p(jnp.exp(-jnp.abs(x)))
        predict = jnp.sum(bce) / E
        reg = (jnp.mean(e0_ref[...] ** 2) + jnp.mean(e1_ref[...] ** 2)
               + jnp.sum(w_ref[...] ** 2) / OUT_DIM)
        out_ref[...] = jnp.reshape(predict + REG_PARAM * reg, (1, 1))

    return pl.pallas_call(
        body,
        out_shape=jax.ShapeDtypeStruct((1, 1), jnp.float32),
    )(scores, labels, emb0, emb1, w_all)


def kernel(embed_nt0, embed_nt1, edges_e0, edges_e1, edges_e2,
           labels_e0, labels_e1, labels_e2, w_e0, w_e1, w_e2):
    e0 = edges_e0.astype(jnp.int32)
    e1 = edges_e1.astype(jnp.int32)
    e2 = edges_e2.astype(jnp.int32)
    w_all = jnp.stack([w_e0[:, 0], w_e1[:, 0], w_e2[:, 0]])
    labels = jnp.stack([labels_e0, labels_e1, labels_e2])
    scores = _sc_scores(embed_nt0, embed_nt1,
                        e0[:, 0], e0[:, 1], e1[:, 0], e1[:, 1],
                        e2[:, 0], e2[:, 1], w_all)
    scores = jnp.reshape(scores, (3, E))
    return _tc_loss(scores, labels, embed_nt0, embed_nt1, w_all)[0, 0]
